# baseline (speedup 1.0000x reference)
import functools

import jax
import jax.numpy as jnp
from jax import lax
from jax.experimental import pallas as pl
from jax.experimental.pallas import tpu as pltpu
from jax.experimental.pallas import tpu_sc as plsc

_LANES = 16
_SUB = 128
_C = 1024


@functools.lru_cache(maxsize=None)
def _build(V, D, N):
    info = plsc.get_sparse_core_info()
    NC, NS = info.num_cores, info.num_subcores
    NW = NC * NS
    per_w = N // NW
    CS = _C // _SUB
    n_chunks = per_w // _C
    per_w_sub = per_w // _SUB
    groups = _C // _LANES

    mesh = plsc.VectorSubcoreMesh(core_axis_name="c", subcore_axis_name="s")

    @functools.partial(
        pl.kernel,
        out_type=jax.ShapeDtypeStruct((N, D), jnp.float32),
        mesh=mesh,
        compiler_params=pltpu.CompilerParams(
            needs_layout_passes=False, use_tc_tiling_on_sc=False),
        scratch_types=[
            pltpu.VMEM((CS, _SUB), jnp.int32),
            pltpu.VMEM((CS, _SUB), jnp.int32),
            pltpu.VMEM((_C, D), jnp.float32),
            pltpu.SemaphoreType.DMA,
        ],
    )
    def run(idx_hbm, table_hbm, out_hbm, raw_v, safe_v, rows_v, sem):
        wid = lax.axis_index("s") * NC + lax.axis_index("c")
        base = wid * per_w
        base_sub = wid * per_w_sub

        def chunk(ci, carry):
            off = base + ci * _C
            pltpu.sync_copy(idx_hbm.at[pl.ds(base_sub + ci * CS, CS)], raw_v)

            def clamp(j, c2):
                def clamp16(i, c3):
                    v = raw_v[j, pl.ds(i * _LANES, _LANES)]
                    safe_v[j, pl.ds(i * _LANES, _LANES)] = jnp.where(v < V, v, 0)
                    return c3
                return lax.fori_loop(0, _SUB // _LANES, clamp16, c2)
            lax.fori_loop(0, CS, clamp, 0)

            copies = [
                pltpu.async_copy(table_hbm.at[safe_v.at[j]],
                                 rows_v.at[pl.ds(j * _SUB, _SUB)], sem)
                for j in range(CS)
            ]
            for cp in copies:
                cp.wait()

            z = jnp.zeros((_LANES,), jnp.float32)

            def fix(g, c2):
                j = g // (_SUB // _LANES)
                i = g % (_SUB // _LANES)
                v = raw_v[j, pl.ds(i * _LANES, _LANES)]
                oov = v >= V
                rid = lax.iota(jnp.int32, _LANES) + g * _LANES
                for col in range(D):
                    cv = jnp.full((_LANES,), col, jnp.int32)
                    plsc.store_scatter(rows_v, [rid, cv], z, mask=oov)
                return c2
            lax.fori_loop(0, groups, fix, 0)

            pltpu.sync_copy(rows_v, out_hbm.at[pl.ds(off, _C)])
            return carry

        lax.fori_loop(0, n_chunks, chunk, 0)

    return run


def kernel(indices, table):
    B, S = indices.shape
    V, D = table.shape
    N = B * S
    idx = indices.reshape(N).astype(jnp.int32).reshape(N // _SUB, _SUB)
    out = _build(V, D, N)(idx, table)
    return out.reshape(B, S, D)


# speedup vs baseline: 1.4215x; 1.4215x over previous
import functools

import jax
import jax.numpy as jnp
from jax import lax
from jax.experimental import pallas as pl
from jax.experimental.pallas import tpu as pltpu
from jax.experimental.pallas import tpu_sc as plsc

_LANES = 16
_R = 32
_FIRE = 8


@functools.lru_cache(maxsize=None)
def _build(B, S, V, D):
    info = plsc.get_sparse_core_info()
    NC, NS = info.num_cores, info.num_subcores
    NW = NC * NS
    rows_w = B // NW
    n_chunks = rows_w // _R
    goffs = list(range(0, S - _LANES + 1, _LANES))
    if goffs[-1] != S - _LANES:
        goffs.append(S - _LANES)

    mesh = plsc.VectorSubcoreMesh(core_axis_name="c", subcore_axis_name="s")

    @functools.partial(
        pl.kernel,
        out_type=jax.ShapeDtypeStruct((B, S, D), jnp.float32),
        mesh=mesh,
        compiler_params=pltpu.CompilerParams(
            needs_layout_passes=False, use_tc_tiling_on_sc=False),
        scratch_types=[
            pltpu.VMEM((_R, S), jnp.int32),
            pltpu.VMEM((_R, S), jnp.int32),
            pltpu.VMEM((_R, S, D), jnp.float32),
            pltpu.SemaphoreType.DMA,
        ],
    )
    def run(idx_hbm, table_hbm, out_hbm, raw_v, safe_v, rows_v, sem):
        wid = lax.axis_index("s") * NC + lax.axis_index("c")
        base = wid * rows_w

        z = jnp.zeros((_LANES,), jnp.float32)
        cols = [jnp.full((_LANES,), c, jnp.int32) for c in range(D)]

        def chunk(ci, carry):
            row0 = base + ci * _R
            pltpu.sync_copy(idx_hbm.at[pl.ds(row0, _R)], raw_v)

            def clamp(r, c2):
                for go in goffs:
                    v = raw_v[r, pl.ds(go, _LANES)]
                    safe_v[r, pl.ds(go, _LANES)] = jnp.where(v < V, v, 0)
                return c2
            lax.fori_loop(0, _R, clamp, 0)

            def gather(rg, c2):
                r0 = rg * _FIRE
                copies = [
                    pltpu.async_copy(table_hbm.at[safe_v.at[r0 + j]],
                                     rows_v.at[r0 + j], sem)
                    for j in range(_FIRE)
                ]
                for cp in copies:
                    cp.wait()
                return c2
            lax.fori_loop(0, _R // _FIRE, gather, 0)

            def fix(r, c2):
                rid = jnp.full((_LANES,), r, jnp.int32)
                for go in goffs:
                    oov = raw_v[r, pl.ds(go, _LANES)] >= V
                    sid = lax.iota(jnp.int32, _LANES) + go
                    for c in range(D):
                        plsc.store_scatter(rows_v, [rid, sid, cols[c]], z,
                                           mask=oov)
                return c2
            lax.fori_loop(0, _R, fix, 0)

            pltpu.sync_copy(rows_v, out_hbm.at[pl.ds(row0, _R)])
            return carry

        lax.fori_loop(0, n_chunks, chunk, 0)

    return run


def kernel(indices, table):
    B, S = indices.shape
    V, D = table.shape
    return _build(B, S, V, D)(indices, table)
